# Initial kernel scaffold; baseline (speedup 1.0000x reference)
import functools

import jax
import jax.numpy as jnp
from jax import lax
from jax.experimental import pallas as pl
from jax.experimental.pallas import tpu as pltpu
from jax.experimental.pallas import tpu_sc as plsc

_NUM_CP = 10
_FEAT = 128
_BATCH = 16384

_NC, _NS, _L = 2, 16, 16
_NW = _NC * _NS
_TOTAL = _BATCH * _FEAT
_PER_TILE = _TOTAL // _NW
_CHUNK = 16384
_NCHUNK = _PER_TILE // _CHUNK
_ROWS = _CHUNK // _FEAT
_SCALE = jnp.float32(_NUM_CP * (1.0 - 1e-06))

_mesh = plsc.VectorSubcoreMesh(
    core_axis_name="c", subcore_axis_name="s", num_cores=_NC, num_subcores=_NS
)


@functools.partial(
    pl.kernel,
    out_type=jax.ShapeDtypeStruct((_TOTAL,), jnp.float32),
    mesh=_mesh,
    scratch_types=[
        pltpu.VMEM((_NUM_CP * _FEAT,), jnp.float32),
        pltpu.VMEM(((_NUM_CP + 1) * _FEAT,), jnp.float32),
        pltpu.VMEM((_CHUNK,), jnp.float32),
        pltpu.VMEM((_CHUNK,), jnp.float32),
    ],
)
def _curve_map_sc(x_hbm, cp_hbm, out_hbm, cp_v, curve_v, in_v, out_v):
    wid = lax.axis_index("s") * _NC + lax.axis_index("c")
    iota = lax.iota(jnp.int32, _L)

    pltpu.sync_copy(cp_hbm, cp_v)
    for c in range(_FEAT // _L):
        col = c * _L
        acc = jnp.zeros((_L,), jnp.float32)
        curve_v[pl.ds(col, _L)] = acc
        for r in range(_NUM_CP):
            acc = acc + jnp.exp(cp_v[pl.ds(r * _FEAT + col, _L)])
            curve_v[pl.ds((r + 1) * _FEAT + col, _L)] = acc
        inv = 1.0 / acc
        for r in range(1, _NUM_CP + 1):
            off = r * _FEAT + col
            curve_v[pl.ds(off, _L)] = curve_v[pl.ds(off, _L)] * inv

    base = wid * _PER_TILE
    for ch in range(_NCHUNK):
        off = base + ch * _CHUNK
        pltpu.sync_copy(x_hbm.at[pl.ds(off, _CHUNK)], in_v)

        def row_body(r, carry):
            for c in range(_FEAT // _L):
                o = r * _FEAT + c * _L
                xv = in_v[pl.ds(o, _L)] * _SCALE
                idx = xv.astype(jnp.int32)
                w = xv - idx.astype(jnp.float32)
                t = idx * _FEAT + (iota + (c * _L))
                lo = plsc.load_gather(curve_v, [t])
                hi = plsc.load_gather(curve_v, [t + _FEAT])
                out_v[pl.ds(o, _L)] = lo + w * (hi - lo)
            return carry

        lax.fori_loop(0, _ROWS, row_body, 0)
        pltpu.sync_copy(out_v, out_hbm.at[pl.ds(off, _CHUNK)])


def kernel(inputs, control_points):
    out = _curve_map_sc(inputs.reshape(-1), control_points.reshape(-1))
    return out.reshape(inputs.shape)


# speedup vs baseline: 669.1423x; 669.1423x over previous
import functools

import jax
import jax.numpy as jnp
from jax import lax
from jax.experimental import pallas as pl
from jax.experimental.pallas import tpu as pltpu
from jax.experimental.pallas import tpu_sc as plsc

_NUM_CP = 10
_FEAT = 128
_BATCH = 16384

_NC, _NS, _L = 2, 16, 16
_NW = _NC * _NS
_TOTAL = _BATCH * _FEAT
_PER_TILE = _TOTAL // _NW
_CHUNK = 16384
_NCHUNK = _PER_TILE // _CHUNK
_ROWS = _CHUNK // _FEAT
_SCALE = float(_NUM_CP * (1.0 - 1e-06))

_mesh = plsc.VectorSubcoreMesh(
    core_axis_name="c", subcore_axis_name="s", num_cores=_NC, num_subcores=_NS
)


@functools.partial(
    pl.kernel,
    out_type=jax.ShapeDtypeStruct((_TOTAL,), jnp.float32),
    mesh=_mesh,
    scratch_types=[
        pltpu.VMEM((_NUM_CP * _FEAT,), jnp.float32),
        pltpu.VMEM(((_NUM_CP + 1) * _FEAT,), jnp.float32),
        pltpu.VMEM((_CHUNK,), jnp.float32),
        pltpu.VMEM((_CHUNK,), jnp.float32),
    ],
    compiler_params=pltpu.CompilerParams(needs_layout_passes=False),
)
def _curve_map_sc(x_hbm, cp_hbm, out_hbm, cp_v, curve_v, in_v, out_v):
    wid = lax.axis_index("s") * _NC + lax.axis_index("c")
    iota = lax.iota(jnp.int32, _L)

    pltpu.sync_copy(cp_hbm, cp_v)
    for c in range(_FEAT // _L):
        col = c * _L
        acc = jnp.zeros((_L,), jnp.float32)
        curve_v[pl.ds(col, _L)] = acc
        for r in range(_NUM_CP):
            acc = acc + jnp.exp(cp_v[pl.ds(r * _FEAT + col, _L)])
            curve_v[pl.ds((r + 1) * _FEAT + col, _L)] = acc
        inv = 1.0 / acc
        for r in range(1, _NUM_CP + 1):
            off = r * _FEAT + col
            curve_v[pl.ds(off, _L)] = curve_v[pl.ds(off, _L)] * inv

    base = wid * _PER_TILE
    for ch in range(_NCHUNK):
        off = base + ch * _CHUNK
        pltpu.sync_copy(x_hbm.at[pl.ds(off, _CHUNK)], in_v)

        def row_body(r, carry):
            for c in range(_FEAT // _L):
                o = r * _FEAT + c * _L
                xv = in_v[pl.ds(o, _L)] * _SCALE
                idx = xv.astype(jnp.int32)
                w = xv - idx.astype(jnp.float32)
                t = idx * _FEAT + (iota + (c * _L))
                lo = plsc.load_gather(curve_v, [t])
                hi = plsc.load_gather(curve_v, [t + _FEAT])
                out_v[pl.ds(o, _L)] = lo + w * (hi - lo)
            return carry

        lax.fori_loop(0, _ROWS, row_body, 0)
        pltpu.sync_copy(out_v, out_hbm.at[pl.ds(off, _CHUNK)])


def kernel(inputs, control_points):
    out = _curve_map_sc(inputs.reshape(-1), control_points.reshape(-1))
    return out.reshape(inputs.shape)


# speedup vs baseline: 1102.4064x; 1.6475x over previous
import functools

import jax
import jax.numpy as jnp
from jax import lax
from jax.experimental import pallas as pl
from jax.experimental.pallas import tpu as pltpu
from jax.experimental.pallas import tpu_sc as plsc

_NUM_CP = 10
_FEAT = 128
_BATCH = 16384

_NC, _NS, _L = 2, 16, 16
_NW = _NC * _NS
_TOTAL = _BATCH * _FEAT
_PER_TILE = _TOTAL // _NW
_CHUNK = 16384
_NCHUNK = _PER_TILE // _CHUNK
_ROWS = _CHUNK // _FEAT
_SCALE = float(_NUM_CP * (1.0 - 1e-06))

_mesh = plsc.VectorSubcoreMesh(
    core_axis_name="c", subcore_axis_name="s", num_cores=_NC, num_subcores=_NS
)


@functools.partial(
    pl.kernel,
    out_type=jax.ShapeDtypeStruct((_TOTAL,), jnp.float32),
    mesh=_mesh,
    scratch_types=[
        pltpu.VMEM((_NUM_CP * _FEAT,), jnp.float32),
        pltpu.VMEM(((_NUM_CP + 1) * _FEAT,), jnp.float32),
        pltpu.VMEM((_CHUNK,), jnp.float32),
        pltpu.VMEM((_CHUNK,), jnp.float32),
        pltpu.VMEM((_CHUNK,), jnp.float32),
        pltpu.VMEM((_CHUNK,), jnp.float32),
        pltpu.SemaphoreType.DMA,
        pltpu.SemaphoreType.DMA,
        pltpu.SemaphoreType.DMA,
        pltpu.SemaphoreType.DMA,
    ],
    compiler_params=pltpu.CompilerParams(needs_layout_passes=False),
)
def _curve_map_sc(x_hbm, cp_hbm, out_hbm, cp_v, curve_v,
                  in_v0, in_v1, out_v0, out_v1, si0, si1, so0, so1):
    wid = lax.axis_index("s") * _NC + lax.axis_index("c")
    iota = lax.iota(jnp.int32, _L)
    in_bufs, out_bufs = (in_v0, in_v1), (out_v0, out_v1)
    in_sems, out_sems = (si0, si1), (so0, so1)
    base = wid * _PER_TILE

    h_in = [None] * _NCHUNK
    h_out = [None] * _NCHUNK
    for ch in range(min(2, _NCHUNK)):
        h_in[ch] = pltpu.async_copy(
            x_hbm.at[pl.ds(base + ch * _CHUNK, _CHUNK)], in_bufs[ch % 2],
            in_sems[ch % 2])

    pltpu.sync_copy(cp_hbm, cp_v)
    for c in range(_FEAT // _L):
        col = c * _L
        acc = jnp.zeros((_L,), jnp.float32)
        curve_v[pl.ds(col, _L)] = acc
        for r in range(_NUM_CP):
            acc = acc + jnp.exp(cp_v[pl.ds(r * _FEAT + col, _L)])
            curve_v[pl.ds((r + 1) * _FEAT + col, _L)] = acc
        inv = 1.0 / acc
        for r in range(1, _NUM_CP + 1):
            off = r * _FEAT + col
            curve_v[pl.ds(off, _L)] = curve_v[pl.ds(off, _L)] * inv

    for ch in range(_NCHUNK):
        cur = ch % 2
        in_v, out_v = in_bufs[cur], out_bufs[cur]
        h_in[ch].wait()
        if ch >= 2:
            h_out[ch - 2].wait()

        @plsc.parallel_loop(0, _ROWS, unroll=2)
        def row_body(r):
            for c in range(_FEAT // _L):
                o = r * _FEAT + c * _L
                xv = in_v[pl.ds(o, _L)] * _SCALE
                idx = xv.astype(jnp.int32)
                w = xv - idx.astype(jnp.float32)
                t = idx * _FEAT + (iota + (c * _L))
                lo = plsc.load_gather(curve_v, [t])
                hi = plsc.load_gather(curve_v, [t + _FEAT])
                out_v[pl.ds(o, _L)] = lo + w * (hi - lo)

        h_out[ch] = pltpu.async_copy(
            out_v, out_hbm.at[pl.ds(base + ch * _CHUNK, _CHUNK)], out_sems[cur])
        if ch + 2 < _NCHUNK:
            h_in[ch + 2] = pltpu.async_copy(
                x_hbm.at[pl.ds(base + (ch + 2) * _CHUNK, _CHUNK)], in_v,
                in_sems[cur])

    for ch in range(max(_NCHUNK - 2, 0), _NCHUNK):
        h_out[ch].wait()


def kernel(inputs, control_points):
    out = _curve_map_sc(inputs.reshape(-1), control_points.reshape(-1))
    return out.reshape(inputs.shape)


# speedup vs baseline: 1146.1994x; 1.0397x over previous
import functools

import jax
import jax.numpy as jnp
from jax import lax
from jax.experimental import pallas as pl
from jax.experimental.pallas import tpu as pltpu
from jax.experimental.pallas import tpu_sc as plsc

_NUM_CP = 10
_FEAT = 128
_BATCH = 16384

_NC, _NS, _L = 2, 16, 16
_NW = _NC * _NS
_TOTAL = _BATCH * _FEAT
_PER_TILE = _TOTAL // _NW
_CHUNK = 16384
_NCHUNK = _PER_TILE // _CHUNK
_ROWS = _CHUNK // _FEAT
_SCALE = float(_NUM_CP * (1.0 - 1e-06))

_mesh = plsc.VectorSubcoreMesh(
    core_axis_name="c", subcore_axis_name="s", num_cores=_NC, num_subcores=_NS
)


@functools.partial(
    pl.kernel,
    out_type=jax.ShapeDtypeStruct((_TOTAL,), jnp.float32),
    mesh=_mesh,
    scratch_types=[
        pltpu.VMEM((_NUM_CP * _FEAT,), jnp.float32),
        pltpu.VMEM(((_NUM_CP + 1) * _FEAT,), jnp.float32),
        pltpu.VMEM((_CHUNK,), jnp.float32),
        pltpu.VMEM((_CHUNK,), jnp.float32),
        pltpu.VMEM((_CHUNK,), jnp.float32),
        pltpu.VMEM((_CHUNK,), jnp.float32),
        pltpu.SemaphoreType.DMA,
        pltpu.SemaphoreType.DMA,
        pltpu.SemaphoreType.DMA,
        pltpu.SemaphoreType.DMA,
    ],
    compiler_params=pltpu.CompilerParams(needs_layout_passes=False),
)
def _curve_map_sc(x_hbm, cp_hbm, out_hbm, cp_v, curve_v,
                  in_v0, in_v1, out_v0, out_v1, si0, si1, so0, so1):
    wid = lax.axis_index("s") * _NC + lax.axis_index("c")
    iota = lax.iota(jnp.int32, _L)
    in_bufs, out_bufs = (in_v0, in_v1), (out_v0, out_v1)
    in_sems, out_sems = (si0, si1), (so0, so1)
    base = wid * _PER_TILE

    h_in = [None] * _NCHUNK
    h_out = [None] * _NCHUNK
    for ch in range(min(2, _NCHUNK)):
        h_in[ch] = pltpu.async_copy(
            x_hbm.at[pl.ds(base + ch * _CHUNK, _CHUNK)], in_bufs[ch % 2],
            in_sems[ch % 2])

    pltpu.sync_copy(cp_hbm, cp_v)
    for c in range(_FEAT // _L):
        col = c * _L
        acc = jnp.zeros((_L,), jnp.float32)
        curve_v[pl.ds(col, _L)] = acc
        for r in range(_NUM_CP):
            acc = acc + jnp.exp(cp_v[pl.ds(r * _FEAT + col, _L)])
            curve_v[pl.ds((r + 1) * _FEAT + col, _L)] = acc
        inv = 1.0 / acc
        for r in range(1, _NUM_CP + 1):
            off = r * _FEAT + col
            curve_v[pl.ds(off, _L)] = curve_v[pl.ds(off, _L)] * inv

    for ch in range(_NCHUNK):
        cur = ch % 2
        in_v, out_v = in_bufs[cur], out_bufs[cur]
        h_in[ch].wait()
        if ch >= 2:
            h_out[ch - 2].wait()

        curve_hi = curve_v.at[pl.ds(_FEAT, _NUM_CP * _FEAT)]

        @plsc.parallel_loop(0, _ROWS, unroll=4)
        def row_body(r):
            for c in range(_FEAT // _L):
                o = r * _FEAT + c * _L
                xv = in_v[pl.ds(o, _L)] * _SCALE
                idx = xv.astype(jnp.int32)
                w = xv - idx.astype(jnp.float32)
                t = idx * _FEAT + (iota + (c * _L))
                lo = plsc.load_gather(curve_v, [t])
                hi = plsc.load_gather(curve_hi, [t])
                out_v[pl.ds(o, _L)] = lo + w * (hi - lo)

        h_out[ch] = pltpu.async_copy(
            out_v, out_hbm.at[pl.ds(base + ch * _CHUNK, _CHUNK)], out_sems[cur])
        if ch + 2 < _NCHUNK:
            h_in[ch + 2] = pltpu.async_copy(
                x_hbm.at[pl.ds(base + (ch + 2) * _CHUNK, _CHUNK)], in_v,
                in_sems[cur])

    for ch in range(max(_NCHUNK - 2, 0), _NCHUNK):
        h_out[ch].wait()


def kernel(inputs, control_points):
    out = _curve_map_sc(inputs.reshape(-1), control_points.reshape(-1))
    return out.reshape(inputs.shape)


# speedup vs baseline: 1226.7599x; 1.0703x over previous
import functools

import jax
import jax.numpy as jnp
from jax import lax
from jax.experimental import pallas as pl
from jax.experimental.pallas import tpu as pltpu
from jax.experimental.pallas import tpu_sc as plsc

_NUM_CP = 10
_FEAT = 128
_BATCH = 16384

_NC, _NS, _L = 2, 16, 16
_NW = _NC * _NS
_TOTAL = _BATCH * _FEAT
_PER_TILE = _TOTAL // _NW
_CHUNK = 32768
_NCHUNK = _PER_TILE // _CHUNK
_ROWS = _CHUNK // _FEAT
_SCALE = float(_NUM_CP * (1.0 - 1e-06))

_mesh = plsc.VectorSubcoreMesh(
    core_axis_name="c", subcore_axis_name="s", num_cores=_NC, num_subcores=_NS
)


@functools.partial(
    pl.kernel,
    out_type=jax.ShapeDtypeStruct((_TOTAL,), jnp.float32),
    mesh=_mesh,
    scratch_types=[
        pltpu.VMEM((_NUM_CP * _FEAT,), jnp.float32),
        pltpu.VMEM(((_NUM_CP + 1) * _FEAT,), jnp.float32),
        pltpu.VMEM((_CHUNK,), jnp.float32),
        pltpu.VMEM((_CHUNK,), jnp.float32),
        pltpu.SemaphoreType.DMA,
        pltpu.SemaphoreType.DMA,
        pltpu.SemaphoreType.DMA,
        pltpu.SemaphoreType.DMA,
    ],
    compiler_params=pltpu.CompilerParams(needs_layout_passes=False),
)
def _curve_map_sc(x_hbm, cp_hbm, out_hbm, cp_v, curve_v,
                  buf0, buf1, si0, si1, so0, so1):
    wid = lax.axis_index("s") * _NC + lax.axis_index("c")
    iota = lax.iota(jnp.int32, _L)
    bufs = (buf0, buf1)
    in_sems, out_sems = (si0, si1), (so0, so1)
    base = wid * _PER_TILE

    h_in = [None] * _NCHUNK
    h_out = [None] * _NCHUNK
    for ch in range(_NCHUNK):
        h_in[ch] = pltpu.async_copy(
            x_hbm.at[pl.ds(base + ch * _CHUNK, _CHUNK)], bufs[ch % 2],
            in_sems[ch % 2])

    pltpu.sync_copy(cp_hbm, cp_v)
    for c in range(_FEAT // _L):
        col = c * _L
        acc = jnp.zeros((_L,), jnp.float32)
        curve_v[pl.ds(col, _L)] = acc
        for r in range(_NUM_CP):
            acc = acc + jnp.exp(cp_v[pl.ds(r * _FEAT + col, _L)])
            curve_v[pl.ds((r + 1) * _FEAT + col, _L)] = acc
        inv = 1.0 / acc
        for r in range(1, _NUM_CP + 1):
            off = r * _FEAT + col
            curve_v[pl.ds(off, _L)] = curve_v[pl.ds(off, _L)] * inv

    curve_hi = curve_v.at[pl.ds(_FEAT, _NUM_CP * _FEAT)]
    for ch in range(_NCHUNK):
        buf = bufs[ch % 2]
        h_in[ch].wait()

        @plsc.parallel_loop(0, _ROWS, unroll=2)
        def row_body(r):
            for c in range(_FEAT // _L):
                o = r * _FEAT + c * _L
                xv = buf[pl.ds(o, _L)] * _SCALE
                idx = xv.astype(jnp.int32)
                w = xv - idx.astype(jnp.float32)
                t = idx * _FEAT + (iota + (c * _L))
                lo = plsc.load_gather(curve_v, [t])
                hi = plsc.load_gather(curve_hi, [t])
                buf[pl.ds(o, _L)] = lo + w * (hi - lo)

        h_out[ch] = pltpu.async_copy(
            buf, out_hbm.at[pl.ds(base + ch * _CHUNK, _CHUNK)], out_sems[ch % 2])

    for ch in range(_NCHUNK):
        h_out[ch].wait()


def kernel(inputs, control_points):
    out = _curve_map_sc(inputs.reshape(-1), control_points.reshape(-1))
    return out.reshape(inputs.shape)


# speedup vs baseline: 1269.0418x; 1.0345x over previous
import functools

import jax
import jax.numpy as jnp
from jax import lax
from jax.experimental import pallas as pl
from jax.experimental.pallas import tpu as pltpu
from jax.experimental.pallas import tpu_sc as plsc

_NUM_CP = 10
_FEAT = 128
_BATCH = 16384

_NC, _NS, _L = 2, 16, 16
_NW = _NC * _NS
_TOTAL = _BATCH * _FEAT
_PER_TILE = _TOTAL // _NW
_CHUNK = 32768
_NCHUNK = _PER_TILE // _CHUNK
_ROWS = _CHUNK // _FEAT
_SCALE = float(_NUM_CP * (1.0 - 1e-06))

_mesh = plsc.VectorSubcoreMesh(
    core_axis_name="c", subcore_axis_name="s", num_cores=_NC, num_subcores=_NS
)


@functools.partial(
    pl.kernel,
    out_type=jax.ShapeDtypeStruct((_TOTAL,), jnp.float32),
    mesh=_mesh,
    scratch_types=[
        pltpu.VMEM((_NUM_CP * _FEAT,), jnp.float32),
        pltpu.VMEM(((_NUM_CP + 1) * _FEAT,), jnp.float32),
        pltpu.VMEM((_CHUNK,), jnp.float32),
        pltpu.VMEM((_CHUNK,), jnp.float32),
        pltpu.SemaphoreType.DMA,
        pltpu.SemaphoreType.DMA,
        pltpu.SemaphoreType.DMA,
        pltpu.SemaphoreType.DMA,
    ],
    compiler_params=pltpu.CompilerParams(needs_layout_passes=False),
)
def _curve_map_sc(x_hbm, cp_hbm, out_hbm, cp_v, curve_v,
                  buf0, buf1, si0, si1, so0, so1):
    wid = lax.axis_index("s") * _NC + lax.axis_index("c")
    iota = lax.iota(jnp.int32, _L)
    bufs = (buf0, buf1)
    in_sems, out_sems = (si0, si1), (so0, so1)
    base = wid * _PER_TILE

    h_in = [None] * _NCHUNK
    h_out = [None] * _NCHUNK
    for ch in range(_NCHUNK):
        h_in[ch] = pltpu.async_copy(
            x_hbm.at[pl.ds(base + ch * _CHUNK, _CHUNK)], bufs[ch % 2],
            in_sems[ch % 2])

    pltpu.sync_copy(cp_hbm, cp_v)

    @plsc.parallel_loop(0, _FEAT // _L)
    def col_body(c):
        col = c * _L
        curve_v[pl.ds(col, _L)] = jnp.zeros((_L,), jnp.float32)

        def cum_body(r, acc):
            acc = acc + jnp.exp(cp_v[pl.ds(r * _FEAT + col, _L)])
            curve_v[pl.ds((r + 1) * _FEAT + col, _L)] = acc
            return acc

        total = lax.fori_loop(0, _NUM_CP, cum_body, jnp.zeros((_L,), jnp.float32))
        inv = 1.0 / total

        def norm_body(r, _):
            off = r * _FEAT + col
            curve_v[pl.ds(off, _L)] = curve_v[pl.ds(off, _L)] * inv
            return 0

        lax.fori_loop(1, _NUM_CP + 1, norm_body, 0)

    curve_hi = curve_v.at[pl.ds(_FEAT, _NUM_CP * _FEAT)]
    for ch in range(_NCHUNK):
        buf = bufs[ch % 2]
        h_in[ch].wait()

        @plsc.parallel_loop(0, _ROWS, unroll=4)
        def row_body(r):
            for c in range(_FEAT // _L):
                o = r * _FEAT + c * _L
                xv = buf[pl.ds(o, _L)] * _SCALE
                idx = xv.astype(jnp.int32)
                w = xv - idx.astype(jnp.float32)
                t = idx * _FEAT + (iota + (c * _L))
                lo = plsc.load_gather(curve_v, [t])
                hi = plsc.load_gather(curve_hi, [t])
                buf[pl.ds(o, _L)] = lo + w * (hi - lo)

        h_out[ch] = pltpu.async_copy(
            buf, out_hbm.at[pl.ds(base + ch * _CHUNK, _CHUNK)], out_sems[ch % 2])

    for ch in range(_NCHUNK):
        h_out[ch].wait()


def kernel(inputs, control_points):
    out = _curve_map_sc(inputs.reshape(-1), control_points.reshape(-1))
    return out.reshape(inputs.shape)
